# baseline (speedup 1.0000x reference)
import math

import jax
import jax.numpy as jnp
import numpy as np
from jax import lax
from jax.experimental import pallas as pl
from jax.experimental.pallas import tpu as pltpu
from jax.experimental.pallas import tpu_sc as plsc

D_MODEL = 2048
VOCAB = 50257
AST_VOCAB = 512
MAX_LEN = 2048
BATCH = 4
SEQ = 2048
N_TOK = BATCH * SEQ
SCALE = math.sqrt(D_MODEL)

NUM_CORES = 2
NUM_SUBCORES = 16
NUM_WORKERS = NUM_CORES * NUM_SUBCORES
S_PER_W = SEQ // NUM_WORKERS
PER_W = N_TOK // NUM_WORKERS
CHUNK = 8
N_CHUNKS = PER_W // CHUNK
LANES = 16
COL_ITERS = D_MODEL // LANES
WPR = D_MODEL // 2


def _pe_np() -> np.ndarray:
    position = np.arange(MAX_LEN)[:, None].astype(np.float32)
    div_term = np.exp(
        np.arange(0, D_MODEL, 2).astype(np.float32)
        * (-math.log(10000.0) / D_MODEL)
    )
    pe = np.zeros((MAX_LEN, D_MODEL), dtype=np.float32)
    pe[:, 0::2] = np.sin(position * div_term)
    pe[:, 1::2] = np.cos(position * div_term)
    return pe


_PE = _pe_np()


def _pack_rows_np(a: np.ndarray) -> np.ndarray:
    r, c = a.shape
    arranged = a.reshape(r, c // 32, 2, 16).transpose(0, 1, 3, 2)
    u16 = np.ascontiguousarray(arranged).astype(jnp.bfloat16).view(np.uint16)
    u16 = u16.reshape(r, c // 2, 2)
    return (u16[..., 0].astype(np.uint32)
            | (u16[..., 1].astype(np.uint32) << 16))


_PE_W = _pack_rows_np(_PE)
_MASK_HI = np.uint32(0xFFFF0000)


def _body(ids_hbm, aids_hbm, tok_hbm, ast_hbm, pe_hbm, out_hbm,
          idx_v, aidx_v, tok_b, ast_b, out_b, pe_buf,
          sem_t0, sem_t1, sem_a0, sem_a1, sem_o0, sem_o1,
          sem_p0, sem_p1):
    sem_t = (sem_t0, sem_t1)
    sem_a = (sem_a0, sem_a1)
    sem_o = (sem_o0, sem_o1)

    wid = lax.axis_index("s") * NUM_CORES + lax.axis_index("c")
    wseq0 = pl.multiple_of(wid * S_PER_W, S_PER_W)

    id_copies = []
    for b in range(BATCH):
        cpi = pltpu.make_async_copy(
            ids_hbm.at[b, pl.ds(wseq0, S_PER_W)],
            idx_v.at[pl.ds(b * S_PER_W, S_PER_W)], sem_t0)
        cpa = pltpu.make_async_copy(
            aids_hbm.at[b, pl.ds(wseq0, S_PER_W)],
            aidx_v.at[pl.ds(b * S_PER_W, S_PER_W)], sem_a0)
        cpi.start()
        cpa.start()
        id_copies.append((cpi, cpa))

    def gathers(t, p):
        b = lax.rem(t, BATCH)
        k = lax.div(t, BATCH)
        ioff = pl.multiple_of(b * S_PER_W + k * CHUNK, CHUNK)
        ct = pltpu.make_async_copy(
            tok_hbm.at[idx_v.at[pl.ds(ioff, CHUNK)]], tok_b.at[p], sem_t[p])
        ca = pltpu.make_async_copy(
            ast_hbm.at[aidx_v.at[pl.ds(ioff, CHUNK)]], ast_b.at[p], sem_a[p])
        return ct, ca

    def out_copy(t, p):
        b = lax.rem(t, BATCH)
        k = lax.div(t, BATCH)
        foff = pl.multiple_of(b * SEQ + wseq0 + k * CHUNK, CHUNK)
        return pltpu.make_async_copy(
            out_b.at[p], out_hbm.at[pl.ds(foff, CHUNK)], sem_o[p])

    def pe_copy(k, sem_p):
        return pltpu.make_async_copy(
            pe_hbm.at[pl.ds(
                pl.multiple_of((wseq0 + k * CHUNK) * WPR, WPR),
                CHUNK * WPR)],
            pe_buf.at[lax.rem(k, 2)], sem_p)

    pe_copy(0, sem_p0).start()
    pe_copy(0, sem_p0).wait()
    pe_copy(1, sem_p1).start()
    for cpi, cpa in id_copies:
        cpi.wait()
        cpa.wait()
    ct0, ca0 = gathers(0, 0)
    ct0.start()
    ca0.start()

    def step(t2, carry):
        for p in range(2):
            t = 2 * t2 + p
            tn = t + 1

            @pl.when(tn < N_CHUNKS)
            def _():
                ct, ca = gathers(tn, 1 - p)
                ct.start()
                ca.start()

            @pl.when((t > 0) & (lax.rem(t, BATCH) == 0))
            def _():
                k = lax.div(t, BATCH)
                sem_k = lax.rem(k, 2)
                pl.when(sem_k == 0)(lambda: pe_copy(k, sem_p0).wait())
                pl.when(sem_k == 1)(lambda: pe_copy(k, sem_p1).wait())

                @pl.when(k + 1 < N_CHUNKS // BATCH)
                def _():
                    pl.when(sem_k == 0)(lambda: pe_copy(k + 1, sem_p1).start())
                    pl.when(sem_k == 1)(lambda: pe_copy(k + 1, sem_p0).start())

            ct, ca = gathers(t, p)
            ct.wait()
            ca.wait()

            @pl.when(t >= 2)
            def _():
                out_copy(t - 2, p).wait()

            kp = lax.rem(lax.div(t, BATCH), 2)

            @plsc.parallel_loop(0, COL_ITERS // 2, 1, unroll=8)
            def _(i):
                cw = pl.multiple_of(i * LANES, LANES)
                c32 = pl.multiple_of(i * 2 * LANES, 2 * LANES)
                for r in range(CHUNK):
                    wa = ast_b[p, r, pl.ds(cw, LANES)]
                    wp = pe_buf[kp, pl.ds(r * WPR + cw, LANES)]
                    a_lo = lax.bitcast_convert_type(wa << 16, jnp.float32)
                    a_hi = lax.bitcast_convert_type(wa & _MASK_HI, jnp.float32)
                    p_lo = lax.bitcast_convert_type(wp << 16, jnp.float32)
                    p_hi = lax.bitcast_convert_type(wp & _MASK_HI, jnp.float32)
                    t0 = tok_b[p, r, pl.ds(c32, LANES)]
                    t1 = tok_b[p, r, pl.ds(c32 + LANES, LANES)]
                    out_b[p, r, pl.ds(c32, LANES)] = t0 * SCALE + (a_lo + p_lo)
                    out_b[p, r, pl.ds(c32 + LANES, LANES)] = (
                        t1 * SCALE + (a_hi + p_hi))
            out_copy(t, p).start()
        return carry

    lax.fori_loop(0, N_CHUNKS // 2, step, 0)
    out_copy(N_CHUNKS - 2, 0).wait()
    out_copy(N_CHUNKS - 1, 1).wait()


@jax.jit
def _encode(ids, aids, tok_table, ast_table, pe):
    mesh = plsc.VectorSubcoreMesh(
        core_axis_name="c", subcore_axis_name="s",
        num_cores=NUM_CORES, num_subcores=NUM_SUBCORES)
    f = pl.kernel(
        _body,
        out_type=jax.ShapeDtypeStruct((N_TOK, D_MODEL), jnp.float32),
        mesh=mesh,
        scratch_types=[
            pltpu.VMEM((PER_W,), jnp.int32),
            pltpu.VMEM((PER_W,), jnp.int32),

            pltpu.VMEM((2, CHUNK, D_MODEL), jnp.float32),
            pltpu.VMEM((2, CHUNK, D_MODEL // 2), jnp.uint32),
            pltpu.VMEM((2, CHUNK, D_MODEL), jnp.float32),
            pltpu.VMEM((2, CHUNK * WPR), jnp.uint32),
        ] + [pltpu.SemaphoreType.DMA] * 8,
    )
    return f(ids, aids, tok_table, ast_table, pe)


_PE_DEV_CACHE = []


def _pe_dev():
    if not _PE_DEV_CACHE:
        _PE_DEV_CACHE.append(jnp.asarray(_PE_W.reshape(-1)))
    return _PE_DEV_CACHE[0]


def kernel(input_ids, ast_ids, token_embedding, ast_embedding):
    ids = (input_ids if input_ids.dtype == jnp.int32
           else input_ids.astype(jnp.int32))
    aids = (ast_ids if ast_ids.dtype == jnp.int32
            else ast_ids.astype(jnp.int32))
    ast_bf = (ast_embedding * SCALE).astype(jnp.bfloat16)
    ast_bf = ast_bf.reshape(AST_VOCAB, D_MODEL // 32, 2, 16).transpose(0, 1, 3, 2)
    ast_w = lax.bitcast_convert_type(ast_bf, jnp.uint32)
    ast_w = ast_w.reshape(AST_VOCAB, D_MODEL // 2)
    out = _encode(ids, aids, token_embedding, ast_w, _pe_dev())
    return out.reshape(BATCH, SEQ, D_MODEL)


# speedup vs baseline: 1.0036x; 1.0036x over previous
import math

import jax
import jax.numpy as jnp
import numpy as np
from jax import lax
from jax.experimental import pallas as pl
from jax.experimental.pallas import tpu as pltpu
from jax.experimental.pallas import tpu_sc as plsc

D_MODEL = 2048
VOCAB = 50257
AST_VOCAB = 512
MAX_LEN = 2048
BATCH = 4
SEQ = 2048
N_TOK = BATCH * SEQ
SCALE = math.sqrt(D_MODEL)

NUM_CORES = 2
NUM_SUBCORES = 16
NUM_WORKERS = NUM_CORES * NUM_SUBCORES
S_PER_W = SEQ // NUM_WORKERS
PER_W = N_TOK // NUM_WORKERS
CHUNK = 8
N_CHUNKS = PER_W // CHUNK
LANES = 16
COL_ITERS = D_MODEL // LANES
WPR = D_MODEL // 2


def _pe_np() -> np.ndarray:
    position = np.arange(MAX_LEN)[:, None].astype(np.float32)
    div_term = np.exp(
        np.arange(0, D_MODEL, 2).astype(np.float32)
        * (-math.log(10000.0) / D_MODEL)
    )
    pe = np.zeros((MAX_LEN, D_MODEL), dtype=np.float32)
    pe[:, 0::2] = np.sin(position * div_term)
    pe[:, 1::2] = np.cos(position * div_term)
    return pe


_PE = _pe_np()


def _pack_rows_np(a: np.ndarray) -> np.ndarray:
    r, c = a.shape
    arranged = a.reshape(r, c // 32, 2, 16).transpose(0, 1, 3, 2)
    u16 = np.ascontiguousarray(arranged).astype(jnp.bfloat16).view(np.uint16)
    u16 = u16.reshape(r, c // 2, 2)
    return (u16[..., 0].astype(np.uint32)
            | (u16[..., 1].astype(np.uint32) << 16))


_PE_W = _pack_rows_np(_PE)
_MASK_HI = np.uint32(0xFFFF0000)


def _body(ids_hbm, aids_hbm, tok_hbm, ast_hbm, pe_hbm, out_hbm,
          idx_v, aidx_v, tok_b, ast_b, out_b, pe_buf,
          sem_t0, sem_t1, sem_a0, sem_a1, sem_o0, sem_o1,
          sem_p0, sem_p1):
    sem_t = (sem_t0, sem_t1)
    sem_a = (sem_a0, sem_a1)
    sem_o = (sem_o0, sem_o1)

    wid = lax.axis_index("s") * NUM_CORES + lax.axis_index("c")
    wseq0 = pl.multiple_of(wid * S_PER_W, S_PER_W)

    id_copies = []
    for b in range(BATCH):
        cpi = pltpu.make_async_copy(
            ids_hbm.at[b, pl.ds(wseq0, S_PER_W)],
            idx_v.at[pl.ds(b * S_PER_W, S_PER_W)], sem_t0)
        cpa = pltpu.make_async_copy(
            aids_hbm.at[b, pl.ds(wseq0, S_PER_W)],
            aidx_v.at[pl.ds(b * S_PER_W, S_PER_W)], sem_a0)
        cpi.start()
        cpa.start()
        id_copies.append((cpi, cpa))

    def gathers(t, p):
        b = lax.rem(t, BATCH)
        k = lax.div(t, BATCH)
        ioff = pl.multiple_of(b * S_PER_W + k * CHUNK, CHUNK)
        ct = pltpu.make_async_copy(
            tok_hbm.at[idx_v.at[pl.ds(ioff, CHUNK)]], tok_b.at[p], sem_t[p])
        ca = pltpu.make_async_copy(
            ast_hbm.at[aidx_v.at[pl.ds(ioff, CHUNK)]], ast_b.at[p], sem_a[p])
        return ct, ca

    def out_copy(t, p):
        b = lax.rem(t, BATCH)
        k = lax.div(t, BATCH)
        foff = pl.multiple_of(b * SEQ + wseq0 + k * CHUNK, CHUNK)
        return pltpu.make_async_copy(
            out_b.at[p], out_hbm.at[pl.ds(foff, CHUNK)], sem_o[p])

    def pe_copy(k, sem_p):
        return pltpu.make_async_copy(
            pe_hbm.at[pl.ds(
                pl.multiple_of((wseq0 + k * CHUNK) * WPR, WPR),
                CHUNK * WPR)],
            pe_buf.at[lax.rem(k, 2)], sem_p)

    pe_copy(0, sem_p0).start()
    pe_copy(0, sem_p0).wait()
    pe_copy(1, sem_p1).start()
    for cpi, cpa in id_copies:
        cpi.wait()
        cpa.wait()
    ct0, ca0 = gathers(0, 0)
    ct0.start()
    ca0.start()

    def step(t2, carry):
        for p in range(2):
            t = 2 * t2 + p
            tn = t + 1

            @pl.when(tn < N_CHUNKS)
            def _():
                ct, ca = gathers(tn, 1 - p)
                ct.start()
                ca.start()

            @pl.when((t > 0) & (lax.rem(t, BATCH) == 0))
            def _():
                k = lax.div(t, BATCH)
                sem_k = lax.rem(k, 2)
                pl.when(sem_k == 0)(lambda: pe_copy(k, sem_p0).wait())
                pl.when(sem_k == 1)(lambda: pe_copy(k, sem_p1).wait())

                @pl.when(k + 1 < N_CHUNKS // BATCH)
                def _():
                    pl.when(sem_k == 0)(lambda: pe_copy(k + 1, sem_p1).start())
                    pl.when(sem_k == 1)(lambda: pe_copy(k + 1, sem_p0).start())

            ct, ca = gathers(t, p)
            ct.wait()
            ca.wait()

            @pl.when(t >= 2)
            def _():
                out_copy(t - 2, p).wait()

            kp = lax.rem(lax.div(t, BATCH), 2)

            @plsc.parallel_loop(0, COL_ITERS // 2, 1, unroll=4)
            def _(i):
                cw = pl.multiple_of(i * LANES, LANES)
                c32 = pl.multiple_of(i * 2 * LANES, 2 * LANES)
                for r in range(CHUNK):
                    wa = ast_b[p, r, pl.ds(cw, LANES)]
                    wp = pe_buf[kp, pl.ds(r * WPR + cw, LANES)]
                    a_lo = lax.bitcast_convert_type(wa << 16, jnp.float32)
                    a_hi = lax.bitcast_convert_type(wa & _MASK_HI, jnp.float32)
                    p_lo = lax.bitcast_convert_type(wp << 16, jnp.float32)
                    p_hi = lax.bitcast_convert_type(wp & _MASK_HI, jnp.float32)
                    t0 = tok_b[p, r, pl.ds(c32, LANES)]
                    t1 = tok_b[p, r, pl.ds(c32 + LANES, LANES)]
                    out_b[p, r, pl.ds(c32, LANES)] = t0 * SCALE + (a_lo + p_lo)
                    out_b[p, r, pl.ds(c32 + LANES, LANES)] = (
                        t1 * SCALE + (a_hi + p_hi))
            out_copy(t, p).start()
        return carry

    lax.fori_loop(0, N_CHUNKS // 2, step, 0)
    out_copy(N_CHUNKS - 2, 0).wait()
    out_copy(N_CHUNKS - 1, 1).wait()


@jax.jit
def _encode(ids, aids, tok_table, ast_table, pe):
    mesh = plsc.VectorSubcoreMesh(
        core_axis_name="c", subcore_axis_name="s",
        num_cores=NUM_CORES, num_subcores=NUM_SUBCORES)
    f = pl.kernel(
        _body,
        out_type=jax.ShapeDtypeStruct((N_TOK, D_MODEL), jnp.float32),
        mesh=mesh,
        scratch_types=[
            pltpu.VMEM((PER_W,), jnp.int32),
            pltpu.VMEM((PER_W,), jnp.int32),

            pltpu.VMEM((2, CHUNK, D_MODEL), jnp.float32),
            pltpu.VMEM((2, CHUNK, D_MODEL // 2), jnp.uint32),
            pltpu.VMEM((2, CHUNK, D_MODEL), jnp.float32),
            pltpu.VMEM((2, CHUNK * WPR), jnp.uint32),
        ] + [pltpu.SemaphoreType.DMA] * 8,
    )
    return f(ids, aids, tok_table, ast_table, pe)


_PE_DEV_CACHE = []


def _pe_dev():
    if not _PE_DEV_CACHE:
        _PE_DEV_CACHE.append(jnp.asarray(_PE_W.reshape(-1)))
    return _PE_DEV_CACHE[0]


def kernel(input_ids, ast_ids, token_embedding, ast_embedding):
    ids = (input_ids if input_ids.dtype == jnp.int32
           else input_ids.astype(jnp.int32))
    aids = (ast_ids if ast_ids.dtype == jnp.int32
            else ast_ids.astype(jnp.int32))
    ast_bf = (ast_embedding * SCALE).astype(jnp.bfloat16)
    ast_bf = ast_bf.reshape(AST_VOCAB, D_MODEL // 32, 2, 16).transpose(0, 1, 3, 2)
    ast_w = lax.bitcast_convert_type(ast_bf, jnp.uint32)
    ast_w = ast_w.reshape(AST_VOCAB, D_MODEL // 2)
    out = _encode(ids, aids, token_embedding, ast_w, _pe_dev())
    return out.reshape(BATCH, SEQ, D_MODEL)
